# Initial kernel scaffold; baseline (speedup 1.0000x reference)
import jax
import jax.numpy as jnp
from jax.experimental import pallas as pl


def kernel(x, edge_index, bn_in_g, bn_in_b, W1, b1, bn1_g, bn1_b, W2, b2, bn2_g, bn2_b, att_w, att_b, Wf, bf, bn_out_g, bn_out_b):
    raise NotImplementedError("write your pallas kernel here")


# speedup vs baseline: 1.8414x; 1.8414x over previous
import functools
import jax
import jax.numpy as jnp
from jax import lax
from jax.experimental import pallas as pl
from jax.experimental.pallas import tpu as pltpu

N = 10000
E = 320000
C = 128
H = 64
EB = 2560
NEB = E // EB

_HI = jax.lax.Precision.HIGHEST


def _p0_body(x_ref, g_ref, b_ref, ws_ref, wb_ref, b1_ref, p_ref, q_ref):
    x = x_ref[...]
    mu = jnp.mean(x, axis=0, keepdims=True)
    var = jnp.mean((x - mu) ** 2, axis=0, keepdims=True)
    xbn = (x - mu) * jax.lax.rsqrt(var + 1e-5) * g_ref[...][None, :] + b_ref[...][None, :]
    p_ref[...] = jnp.dot(xbn, ws_ref[...], preferred_element_type=jnp.float32,
                         precision=_HI) + b1_ref[...][None, :]
    q_ref[...] = jnp.dot(xbn, wb_ref[...], preferred_element_type=jnp.float32,
                         precision=_HI)


def _phase0(x, bn_in_g, bn_in_b, W1, b1):
    Wa = W1[:, :C]
    Wb = W1[:, C:]
    WsT = (Wa + Wb).T
    WbT = Wb.T
    return pl.pallas_call(
        _p0_body,
        out_shape=[jax.ShapeDtypeStruct((N, H), jnp.float32),
                   jax.ShapeDtypeStruct((N, H), jnp.float32)],
    )(x, bn_in_g, bn_in_b, WsT, WbT, b1)


def _p2_body(h1_ref, w_ref, c_ref, h2_ref, st_ref):
    h2 = jnp.maximum(jnp.dot(h1_ref[...], w_ref[...],
                             preferred_element_type=jnp.float32,
                             precision=_HI) + c_ref[...][None, :], 0.0)
    h2_ref[...] = h2
    s1 = jnp.sum(h2, axis=0, keepdims=True)
    s2 = jnp.sum(h2 * h2, axis=0, keepdims=True)
    st_ref[...] = jnp.concatenate(
        [s1, s2, jnp.zeros((6, H), jnp.float32)], axis=0)[None]


def _phase2(h1, W2dT, c2):
    return pl.pallas_call(
        _p2_body,
        grid=(NEB,),
        in_specs=[pl.BlockSpec((EB, H), lambda i: (i, 0)),
                  pl.BlockSpec((H, H), lambda i: (0, 0)),
                  pl.BlockSpec((H,), lambda i: (0,))],
        out_specs=[pl.BlockSpec((EB, H), lambda i: (i, 0)),
                   pl.BlockSpec((1, 8, H), lambda i: (i, 0, 0))],
        out_shape=[jax.ShapeDtypeStruct((E, H), jnp.float32),
                   jax.ShapeDtypeStruct((NEB, 8, H), jnp.float32)],
    )(h1, W2dT, c2)


def _p2b_body(h2_ref, wt_ref, l_ref, mx_ref):
    h3 = h2_ref[...].reshape(EB // 128, 128, H)
    l = jnp.sum(h3 * wt_ref[...][None, None, :], axis=-1)
    l_ref[...] = l[None]
    mx_ref[...] = jnp.full((1, 8, H), jnp.max(l), jnp.float32)


def _phase2b(h2, wt):
    return pl.pallas_call(
        _p2b_body,
        grid=(NEB,),
        in_specs=[pl.BlockSpec((EB, H), lambda i: (i, 0)),
                  pl.BlockSpec((H,), lambda i: (0,))],
        out_specs=[pl.BlockSpec((1, EB // 128, 128), lambda i: (i, 0, 0)),
                   pl.BlockSpec((1, 8, H), lambda i: (i, 0, 0))],
        out_shape=[jax.ShapeDtypeStruct((NEB, EB // 128, 128), jnp.float32),
                   jax.ShapeDtypeStruct((NEB, 8, H), jnp.float32)],
    )(h2, wt)


def _p3_body(l_ref, k_ref, e_ref):
    e_ref[...] = jnp.exp(l_ref[...] - k_ref[0, 0])


def _phase3(logits, K):
    return pl.pallas_call(
        _p3_body,
        grid=(NEB,),
        in_specs=[pl.BlockSpec((1, EB // 128, 128), lambda i: (i, 0, 0)),
                  pl.BlockSpec((1, 1), lambda i: (0, 0))],
        out_specs=pl.BlockSpec((1, EB // 128, 128), lambda i: (i, 0, 0)),
        out_shape=jax.ShapeDtypeStruct((NEB, EB // 128, 128), jnp.float32),
    )(logits, K.reshape(1, 1))


def _p5_body(m_ref, a_ref, s_ref, wf_ref, bf_ref, g_ref, b_ref, o_ref):
    M = m_ref[...]
    M = jnp.where(M == -jnp.inf, 0.0, M)
    att = a_ref[...] / (s_ref[...][:, None] + 1e-16)
    out = (jnp.dot(M, wf_ref[...][:, :H].T, preferred_element_type=jnp.float32,
                   precision=_HI)
           + jnp.dot(att, wf_ref[...][:, H:].T, preferred_element_type=jnp.float32,
                     precision=_HI) + bf_ref[...][None, :])
    mu = jnp.mean(out, axis=0, keepdims=True)
    var = jnp.mean((out - mu) ** 2, axis=0, keepdims=True)
    o_ref[...] = jnp.maximum(
        (out - mu) * jax.lax.rsqrt(var + 1e-5) * g_ref[...][None, :]
        + b_ref[...][None, :], 0.0)


def _phase5(M, A, s, Wf, bf, bn_out_g, bn_out_b):
    return pl.pallas_call(
        _p5_body,
        out_shape=jax.ShapeDtypeStruct((N, H), jnp.float32),
    )(M, A, s, Wf, bf, bn_out_g, bn_out_b)


def kernel(x, edge_index, bn_in_g, bn_in_b, W1, b1, bn1_g, bn1_b, W2, b2,
           bn2_g, bn2_b, att_w, att_b, Wf, bf, bn_out_g, bn_out_b):
    row = edge_index[0]
    col = edge_index[1]

    P, Q = _phase0(x, bn_in_g, bn_in_b, W1, b1)

    h1 = jnp.maximum(P[row] - Q[col], 0.0)
    s1 = jnp.sum(h1, axis=0)
    s2 = jnp.sum(h1 * h1, axis=0)

    mu1 = s1 / E
    var1 = s2 / E - mu1 * mu1
    rs1 = bn1_g * jax.lax.rsqrt(var1 + 1e-5)
    W2dT = (W2 * rs1[None, :]).T
    c2 = b2 + (bn1_b - mu1 * rs1) @ W2.T

    h2, st2 = _phase2(h1, W2dT, c2)
    mu2 = jnp.sum(st2[:, 0, :], axis=0) / E
    var2 = jnp.sum(st2[:, 1, :], axis=0) / E - mu2 * mu2
    rs2 = bn2_g * jax.lax.rsqrt(var2 + 1e-5)
    sh2 = bn2_b - mu2 * rs2

    wt = att_w[0] * rs2
    ct = att_b[0] + att_w[0] @ sh2
    logits, mx = _phase2b(h2, wt)
    K = jnp.max(mx[:, 0, 0]) + ct

    e = _phase3(logits + ct, K).reshape(E)

    h2bn = h2 * rs2[None, :] + sh2[None, :]
    M = jax.ops.segment_max(h2bn, col, num_segments=N)
    s = jax.ops.segment_sum(e, col, num_segments=N)
    A = jax.ops.segment_sum(h2bn * e[:, None], col, num_segments=N)

    return _phase5(M, A, s, Wf, bf, bn_out_g, bn_out_b)


# speedup vs baseline: 2.6657x; 1.4476x over previous
import functools
import jax
import jax.numpy as jnp
from jax import lax
from jax.experimental import pallas as pl
from jax.experimental.pallas import tpu as pltpu
from jax.experimental.pallas import tpu_sc as plsc

N = 10000
E = 320000
C = 128
H = 64
EB = 2560
NEB = E // EB

NC = 2
NS = 16
NW = NC * NS
EPW = E // NW
CH = 400
NCH = EPW // CH
GSUB = 5
GLEN = CH // GSUB

_HI = jax.lax.Precision.HIGHEST


def _p0_body(x_ref, g_ref, b_ref, ws_ref, wb_ref, b1_ref, p_ref, q_ref):
    x = x_ref[...]
    mu = jnp.mean(x, axis=0, keepdims=True)
    var = jnp.mean((x - mu) ** 2, axis=0, keepdims=True)
    xbn = (x - mu) * jax.lax.rsqrt(var + 1e-5) * g_ref[...][None, :] + b_ref[...][None, :]
    p_ref[...] = jnp.dot(xbn, ws_ref[...], preferred_element_type=jnp.float32,
                         precision=_HI) + b1_ref[...][None, :]
    q_ref[...] = jnp.dot(xbn, wb_ref[...], preferred_element_type=jnp.float32,
                         precision=_HI)


def _phase0(x, bn_in_g, bn_in_b, W1, b1):
    Wa = W1[:, :C]
    Wb = W1[:, C:]
    WsT = (Wa + Wb).T
    WbT = Wb.T
    return pl.pallas_call(
        _p0_body,
        out_shape=[jax.ShapeDtypeStruct((N, H), jnp.float32),
                   jax.ShapeDtypeStruct((N, H), jnp.float32)],
    )(x, bn_in_g, bn_in_b, WsT, WbT, b1)


def _p1_body(p_hbm, q_hbm, row_hbm, col_hbm, h1_hbm, st_hbm,
             idxr, idxc, pbuf, qbuf, hbuf, stv, semr, semc):
    wid = lax.axis_index("s") * NC + lax.axis_index("c")
    ebase = wid * EPW

    def chunk(ci, acc):
        base = ebase + ci * CH
        pltpu.sync_copy(row_hbm.at[pl.ds(base, CH)], idxr)
        pltpu.sync_copy(col_hbm.at[pl.ds(base, CH)], idxc)
        cps = []
        for j in range(GSUB):
            sl = pl.ds(j * GLEN, GLEN)
            cps.append(pltpu.async_copy(p_hbm.at[idxr.at[sl]], pbuf.at[sl], semr))
            cps.append(pltpu.async_copy(q_hbm.at[idxc.at[sl]], qbuf.at[sl], semc))
        for cp in cps:
            cp.wait()

        def rowfn(r, acc):
            out = []
            for k in range(4):
                sl = pl.ds(k * 16, 16)
                h = jnp.maximum(pbuf[r, sl] - qbuf[r, sl], 0.0)
                hbuf[r, sl] = h
                out.append(acc[k] + h)
                out.append(acc[4 + k] + h * h)
            return (out[0], out[2], out[4], out[6], out[1], out[3], out[5], out[7])

        acc = lax.fori_loop(0, CH, rowfn, acc)
        pltpu.sync_copy(hbuf, h1_hbm.at[pl.ds(base, CH)])
        return acc

    zero = jnp.zeros((16,), jnp.float32)
    acc = lax.fori_loop(0, NCH, chunk, (zero,) * 8)
    for k in range(4):
        stv[pl.ds(k * 16, 16)] = acc[k]
        stv[pl.ds(64 + k * 16, 16)] = acc[4 + k]
    pltpu.sync_copy(stv, st_hbm.at[wid])


def _phase1(P, Q, row, col):
    return pl.kernel(
        _p1_body,
        out_type=[jax.ShapeDtypeStruct((E, H), jnp.float32),
                  jax.ShapeDtypeStruct((NW, 2 * H), jnp.float32)],
        mesh=plsc.VectorSubcoreMesh(core_axis_name="c", subcore_axis_name="s"),
        compiler_params=pltpu.CompilerParams(use_tc_tiling_on_sc=False),
        scratch_types=[
            pltpu.VMEM((CH,), jnp.int32),
            pltpu.VMEM((CH,), jnp.int32),
            pltpu.VMEM((CH, H), jnp.float32),
            pltpu.VMEM((CH, H), jnp.float32),
            pltpu.VMEM((CH, H), jnp.float32),
            pltpu.VMEM((2 * H,), jnp.float32),
            pltpu.SemaphoreType.DMA,
            pltpu.SemaphoreType.DMA,
        ],
    )(P, Q, row, col)


def _p2_body(h1_ref, w_ref, c_ref, h2_ref, st_ref):
    h2 = jnp.maximum(jnp.dot(h1_ref[...], w_ref[...],
                             preferred_element_type=jnp.float32,
                             precision=_HI) + c_ref[...][None, :], 0.0)
    h2_ref[...] = h2
    s1 = jnp.sum(h2, axis=0, keepdims=True)
    s2 = jnp.sum(h2 * h2, axis=0, keepdims=True)
    st_ref[...] = jnp.concatenate(
        [s1, s2, jnp.zeros((6, H), jnp.float32)], axis=0)[None]


def _phase2(h1, W2dT, c2):
    return pl.pallas_call(
        _p2_body,
        grid=(NEB,),
        in_specs=[pl.BlockSpec((EB, H), lambda i: (i, 0)),
                  pl.BlockSpec((H, H), lambda i: (0, 0)),
                  pl.BlockSpec((H,), lambda i: (0,))],
        out_specs=[pl.BlockSpec((EB, H), lambda i: (i, 0)),
                   pl.BlockSpec((1, 8, H), lambda i: (i, 0, 0))],
        out_shape=[jax.ShapeDtypeStruct((E, H), jnp.float32),
                   jax.ShapeDtypeStruct((NEB, 8, H), jnp.float32)],
    )(h1, W2dT, c2)


def _p2b_body(h2_ref, wt_ref, l_ref, mx_ref):
    h3 = h2_ref[...].reshape(EB // 128, 128, H)
    l = jnp.sum(h3 * wt_ref[...][None, None, :], axis=-1)
    l_ref[...] = l[None]
    mx_ref[...] = jnp.full((1, 8, H), jnp.max(l), jnp.float32)


def _phase2b(h2, wt):
    return pl.pallas_call(
        _p2b_body,
        grid=(NEB,),
        in_specs=[pl.BlockSpec((EB, H), lambda i: (i, 0)),
                  pl.BlockSpec((H,), lambda i: (0,))],
        out_specs=[pl.BlockSpec((1, EB // 128, 128), lambda i: (i, 0, 0)),
                   pl.BlockSpec((1, 8, H), lambda i: (i, 0, 0))],
        out_shape=[jax.ShapeDtypeStruct((NEB, EB // 128, 128), jnp.float32),
                   jax.ShapeDtypeStruct((NEB, 8, H), jnp.float32)],
    )(h2, wt)


def _p3_body(l_ref, k_ref, e_ref):
    e_ref[...] = jnp.exp(l_ref[...] - k_ref[0, 0])


def _phase3(logits, K):
    return pl.pallas_call(
        _p3_body,
        grid=(NEB,),
        in_specs=[pl.BlockSpec((1, EB // 128, 128), lambda i: (i, 0, 0)),
                  pl.BlockSpec((1, 1), lambda i: (0, 0))],
        out_specs=pl.BlockSpec((1, EB // 128, 128), lambda i: (i, 0, 0)),
        out_shape=jax.ShapeDtypeStruct((NEB, EB // 128, 128), jnp.float32),
    )(logits, K.reshape(1, 1))


def _p5_body(m_ref, a_ref, s_ref, wf_ref, bf_ref, g_ref, b_ref, o_ref):
    M = m_ref[...]
    M = jnp.where(M == -jnp.inf, 0.0, M)
    att = a_ref[...] / (s_ref[...][:, None] + 1e-16)
    out = (jnp.dot(M, wf_ref[...][:, :H].T, preferred_element_type=jnp.float32,
                   precision=_HI)
           + jnp.dot(att, wf_ref[...][:, H:].T, preferred_element_type=jnp.float32,
                     precision=_HI) + bf_ref[...][None, :])
    mu = jnp.mean(out, axis=0, keepdims=True)
    var = jnp.mean((out - mu) ** 2, axis=0, keepdims=True)
    o_ref[...] = jnp.maximum(
        (out - mu) * jax.lax.rsqrt(var + 1e-5) * g_ref[...][None, :]
        + b_ref[...][None, :], 0.0)


def _phase5(M, A, s, Wf, bf, bn_out_g, bn_out_b):
    return pl.pallas_call(
        _p5_body,
        out_shape=jax.ShapeDtypeStruct((N, H), jnp.float32),
    )(M, A, s, Wf, bf, bn_out_g, bn_out_b)


def kernel(x, edge_index, bn_in_g, bn_in_b, W1, b1, bn1_g, bn1_b, W2, b2,
           bn2_g, bn2_b, att_w, att_b, Wf, bf, bn_out_g, bn_out_b):
    row = edge_index[0]
    col = edge_index[1]

    P, Q = _phase0(x, bn_in_g, bn_in_b, W1, b1)

    h1, st1 = _phase1(P, Q, row, col)
    s1 = jnp.sum(st1[:, :H], axis=0)
    s2 = jnp.sum(st1[:, H:], axis=0)

    mu1 = s1 / E
    var1 = s2 / E - mu1 * mu1
    rs1 = bn1_g * jax.lax.rsqrt(var1 + 1e-5)
    W2dT = (W2 * rs1[None, :]).T
    c2 = b2 + (bn1_b - mu1 * rs1) @ W2.T

    h2, st2 = _phase2(h1, W2dT, c2)
    mu2 = jnp.sum(st2[:, 0, :], axis=0) / E
    var2 = jnp.sum(st2[:, 1, :], axis=0) / E - mu2 * mu2
    rs2 = bn2_g * jax.lax.rsqrt(var2 + 1e-5)
    sh2 = bn2_b - mu2 * rs2

    wt = att_w[0] * rs2
    ct = att_b[0] + att_w[0] @ sh2
    logits, mx = _phase2b(h2, wt)
    K = jnp.max(mx[:, 0, 0]) + ct

    e = _phase3(logits + ct, K).reshape(E)

    h2bn = h2 * rs2[None, :] + sh2[None, :]
    M = jax.ops.segment_max(h2bn, col, num_segments=N)
    s = jax.ops.segment_sum(e, col, num_segments=N)
    A = jax.ops.segment_sum(h2bn * e[:, None], col, num_segments=N)

    return _phase5(M, A, s, Wf, bf, bn_out_g, bn_out_b)


# speedup vs baseline: 2.9872x; 1.1206x over previous
import functools
import jax
import jax.numpy as jnp
from jax import lax
from jax.experimental import pallas as pl
from jax.experimental.pallas import tpu as pltpu
from jax.experimental.pallas import tpu_sc as plsc

N = 10000
E = 320000
C = 128
H = 64
EB = 2560
NEB = E // EB

NC = 2
NS = 16
NW = NC * NS
EPW = E // NW
CH = 400
NCH = EPW // CH
GSUB = 5
GLEN = CH // GSUB

_HI = jax.lax.Precision.HIGHEST


def _p0_body(x_ref, g_ref, b_ref, ws_ref, wb_ref, b1_ref, p_ref, q_ref):
    x = x_ref[...]
    mu = jnp.mean(x, axis=0, keepdims=True)
    var = jnp.mean((x - mu) ** 2, axis=0, keepdims=True)
    xbn = (x - mu) * jax.lax.rsqrt(var + 1e-5) * g_ref[...][None, :] + b_ref[...][None, :]
    p_ref[...] = jnp.dot(xbn, ws_ref[...], preferred_element_type=jnp.float32,
                         precision=_HI) + b1_ref[...][None, :]
    q_ref[...] = jnp.dot(xbn, wb_ref[...], preferred_element_type=jnp.float32,
                         precision=_HI)


def _phase0(x, bn_in_g, bn_in_b, W1, b1):
    Wa = W1[:, :C]
    Wb = W1[:, C:]
    WsT = (Wa + Wb).T
    WbT = Wb.T
    return pl.pallas_call(
        _p0_body,
        out_shape=[jax.ShapeDtypeStruct((N, H), jnp.float32),
                   jax.ShapeDtypeStruct((N, H), jnp.float32)],
    )(x, bn_in_g, bn_in_b, WsT, WbT, b1)


def _p1_body(p_hbm, q_hbm, row_hbm, col_hbm, h1_hbm, st_hbm,
             idxr, idxc, pbuf, qbuf, hbuf, stv, semr, semc):
    wid = lax.axis_index("s") * NC + lax.axis_index("c")
    ebase = wid * EPW

    def chunk(ci, acc):
        base = ebase + ci * CH
        pltpu.sync_copy(row_hbm.at[pl.ds(base, CH)], idxr)
        pltpu.sync_copy(col_hbm.at[pl.ds(base, CH)], idxc)
        cps = []
        for j in range(GSUB):
            sl = pl.ds(j * GLEN, GLEN)
            cps.append(pltpu.async_copy(p_hbm.at[idxr.at[sl]], pbuf.at[sl], semr))
            cps.append(pltpu.async_copy(q_hbm.at[idxc.at[sl]], qbuf.at[sl], semc))
        for cp in cps:
            cp.wait()

        def rowfn(r, acc):
            out = []
            for k in range(4):
                sl = pl.ds(k * 16, 16)
                h = jnp.maximum(pbuf[r, sl] - qbuf[r, sl], 0.0)
                hbuf[r, sl] = h
                out.append(acc[k] + h)
                out.append(acc[4 + k] + h * h)
            return (out[0], out[2], out[4], out[6], out[1], out[3], out[5], out[7])

        acc = lax.fori_loop(0, CH, rowfn, acc)
        pltpu.sync_copy(hbuf, h1_hbm.at[pl.ds(base, CH)])
        return acc

    zero = jnp.zeros((16,), jnp.float32)
    acc = lax.fori_loop(0, NCH, chunk, (zero,) * 8)
    for k in range(4):
        stv[pl.ds(k * 16, 16)] = acc[k]
        stv[pl.ds(64 + k * 16, 16)] = acc[4 + k]
    pltpu.sync_copy(stv, st_hbm.at[wid])


def _phase1(P, Q, row, col):
    return pl.kernel(
        _p1_body,
        out_type=[jax.ShapeDtypeStruct((E, H), jnp.float32),
                  jax.ShapeDtypeStruct((NW, 2 * H), jnp.float32)],
        mesh=plsc.VectorSubcoreMesh(core_axis_name="c", subcore_axis_name="s"),
        compiler_params=pltpu.CompilerParams(use_tc_tiling_on_sc=False),
        scratch_types=[
            pltpu.VMEM((CH,), jnp.int32),
            pltpu.VMEM((CH,), jnp.int32),
            pltpu.VMEM((CH, H), jnp.float32),
            pltpu.VMEM((CH, H), jnp.float32),
            pltpu.VMEM((CH, H), jnp.float32),
            pltpu.VMEM((2 * H,), jnp.float32),
            pltpu.SemaphoreType.DMA,
            pltpu.SemaphoreType.DMA,
        ],
    )(P, Q, row, col)


def _p2_body(h1_ref, w_ref, c_ref, h2_ref, st_ref):
    h2 = jnp.maximum(jnp.dot(h1_ref[...], w_ref[...],
                             preferred_element_type=jnp.float32,
                             precision=_HI) + c_ref[...][None, :], 0.0)
    h2_ref[...] = h2
    s1 = jnp.sum(h2, axis=0, keepdims=True)
    s2 = jnp.sum(h2 * h2, axis=0, keepdims=True)
    st_ref[...] = jnp.concatenate(
        [s1, s2, jnp.zeros((6, H), jnp.float32)], axis=0)[None]


def _phase2(h1, W2dT, c2):
    return pl.pallas_call(
        _p2_body,
        grid=(NEB,),
        in_specs=[pl.BlockSpec((EB, H), lambda i: (i, 0)),
                  pl.BlockSpec((H, H), lambda i: (0, 0)),
                  pl.BlockSpec((H,), lambda i: (0,))],
        out_specs=[pl.BlockSpec((EB, H), lambda i: (i, 0)),
                   pl.BlockSpec((1, 8, H), lambda i: (i, 0, 0))],
        out_shape=[jax.ShapeDtypeStruct((E, H), jnp.float32),
                   jax.ShapeDtypeStruct((NEB, 8, H), jnp.float32)],
    )(h1, W2dT, c2)


def _p2b_body(h2_ref, wt_ref, l_ref, mx_ref):
    h3 = h2_ref[...].reshape(EB // 128, 128, H)
    l = jnp.sum(h3 * wt_ref[...][None, None, :], axis=-1)
    l_ref[...] = l[None]
    mx_ref[...] = jnp.full((1, 8, H), jnp.max(l), jnp.float32)


def _phase2b(h2, wt):
    return pl.pallas_call(
        _p2b_body,
        grid=(NEB,),
        in_specs=[pl.BlockSpec((EB, H), lambda i: (i, 0)),
                  pl.BlockSpec((H,), lambda i: (0,))],
        out_specs=[pl.BlockSpec((1, EB // 128, 128), lambda i: (i, 0, 0)),
                   pl.BlockSpec((1, 8, H), lambda i: (i, 0, 0))],
        out_shape=[jax.ShapeDtypeStruct((NEB, EB // 128, 128), jnp.float32),
                   jax.ShapeDtypeStruct((NEB, 8, H), jnp.float32)],
    )(h2, wt)


def _p3_body(l_ref, k_ref, e_ref):
    e_ref[...] = jnp.exp(l_ref[...] - k_ref[0, 0])


def _phase3(logits, K):
    return pl.pallas_call(
        _p3_body,
        grid=(NEB,),
        in_specs=[pl.BlockSpec((1, EB // 128, 128), lambda i: (i, 0, 0)),
                  pl.BlockSpec((1, 1), lambda i: (0, 0))],
        out_specs=pl.BlockSpec((1, EB // 128, 128), lambda i: (i, 0, 0)),
        out_shape=jax.ShapeDtypeStruct((NEB, EB // 128, 128), jnp.float32),
    )(logits, K.reshape(1, 1))


NPB = 313
NPAD = NPB * NW
SCH = 800
NG = SCH // 16
NSCH = E // SCH
GMAX = SCH + 16


def _p4_body(h2_hbm, col_hbm, e_hbm, aff_hbm, m_hbm, a_hbm, s_hbm,
             colv, ev, idl, cll, ell, gbuf, macc, aacc, sacc, affv, semg):
    wid = lax.axis_index("s") * NC + lax.axis_index("c")
    nlo = wid * NPB
    iota = lax.iota(jnp.int32, 16)

    pltpu.sync_copy(aff_hbm, affv)
    rs = [affv[pl.ds(k * 16, 16)] for k in range(4)]
    sh = [affv[pl.ds(H + k * 16, 16)] for k in range(4)]

    neg = jnp.full((16,), -jnp.inf, jnp.float32)
    zf = jnp.zeros((16,), jnp.float32)
    zi = jnp.zeros((16,), jnp.int32)

    def initrow(i, _):
        for k in range(4):
            sl = pl.ds(k * 16, 16)
            macc[i, sl] = neg
            aacc[i, sl] = zf
        return 0
    lax.fori_loop(0, NPB, initrow, 0)

    def inits(i, _):
        sacc[i, :] = zf
        return 0
    lax.fori_loop(0, NPB, inits, 0)
    for i in range(GMAX // 16):
        idl[pl.ds(i * 16, 16)] = zi

    def chunk(ci, _):
        base = ci * SCH
        pltpu.sync_copy(col_hbm.at[pl.ds(base, SCH)], colv)
        pltpu.sync_copy(e_hbm.at[pl.ds(base, SCH)], ev)

        def grp(g, cnt):
            sl16 = pl.ds(g * 16, 16)
            c16 = colv[sl16]
            msk = (c16 >= nlo) & (c16 < nlo + NPB)
            dst = pl.ds(cnt, 16)
            plsc.store_compressed(idl.at[dst], iota + (base + g * 16), mask=msk)
            plsc.store_compressed(cll.at[dst], c16, mask=msk)
            plsc.store_compressed(ell.at[dst], ev[sl16], mask=msk)
            return cnt + plsc.all_reduce_population_count(msk)[0]
        m = lax.fori_loop(0, NG, grp, 0)
        nb = lax.div(m + 15, 16)

        def fire(j, _):
            sl = pl.ds(pl.multiple_of(j * 16, 16), 16)
            pltpu.async_copy(h2_hbm.at[idl.at[sl]], gbuf.at[sl], semg)
            return 0
        lax.fori_loop(0, nb, fire, 0)

        def drain(j, _):
            sl = pl.ds(pl.multiple_of(j * 16, 16), 16)
            pltpu.make_async_copy(h2_hbm.at[pl.ds(0, 16)], gbuf.at[sl], semg).wait()
            return 0
        lax.fori_loop(0, nb, drain, 0)

        def edge(r, _):
            ln = cll[pl.ds(r, 16)][0] - nlo
            ee = ell[pl.ds(r, 16)][0]
            for k in range(4):
                sl = pl.ds(k * 16, 16)
                v = gbuf[r, sl] * rs[k] + sh[k]
                macc[ln, sl] = jnp.maximum(macc[ln, sl], v)
                aacc[ln, sl] = aacc[ln, sl] + v * ee
            sacc[ln, :] = sacc[ln, :] + ee
            return 0
        lax.fori_loop(0, m, edge, 0)
        return 0

    lax.fori_loop(0, NSCH, chunk, 0)

    pltpu.sync_copy(macc, m_hbm.at[pl.ds(nlo, NPB)])
    pltpu.sync_copy(aacc, a_hbm.at[pl.ds(nlo, NPB)])
    pltpu.sync_copy(sacc, s_hbm.at[wid])


def _phase4(h2, col, e, aff):
    return pl.kernel(
        _p4_body,
        out_type=[jax.ShapeDtypeStruct((NPAD, H), jnp.float32),
                  jax.ShapeDtypeStruct((NPAD, H), jnp.float32),
                  jax.ShapeDtypeStruct((NW, NPB, 16), jnp.float32)],
        mesh=plsc.VectorSubcoreMesh(core_axis_name="c", subcore_axis_name="s"),
        compiler_params=pltpu.CompilerParams(use_tc_tiling_on_sc=False,
                                             needs_layout_passes=False),
        scratch_types=[
            pltpu.VMEM((SCH,), jnp.int32),
            pltpu.VMEM((SCH,), jnp.float32),
            pltpu.VMEM((GMAX,), jnp.int32),
            pltpu.VMEM((GMAX,), jnp.int32),
            pltpu.VMEM((GMAX,), jnp.float32),
            pltpu.VMEM((GMAX, H), jnp.float32),
            pltpu.VMEM((NPB, H), jnp.float32),
            pltpu.VMEM((NPB, H), jnp.float32),
            pltpu.VMEM((NPB, 16), jnp.float32),
            pltpu.VMEM((2 * H,), jnp.float32),
            pltpu.SemaphoreType.DMA,
        ],
    )(h2, col, e, aff)


def _p5_body(m_ref, a_ref, s_ref, wf_ref, bf_ref, g_ref, b_ref, o_ref):
    M = m_ref[...]
    M = jnp.where(M == -jnp.inf, 0.0, M)
    att = a_ref[...] / (s_ref[...][:, None] + 1e-16)
    out = (jnp.dot(M, wf_ref[...][:, :H].T, preferred_element_type=jnp.float32,
                   precision=_HI)
           + jnp.dot(att, wf_ref[...][:, H:].T, preferred_element_type=jnp.float32,
                     precision=_HI) + bf_ref[...][None, :])
    mu = jnp.mean(out, axis=0, keepdims=True)
    var = jnp.mean((out - mu) ** 2, axis=0, keepdims=True)
    o_ref[...] = jnp.maximum(
        (out - mu) * jax.lax.rsqrt(var + 1e-5) * g_ref[...][None, :]
        + b_ref[...][None, :], 0.0)


def _phase5(M, A, s, Wf, bf, bn_out_g, bn_out_b):
    return pl.pallas_call(
        _p5_body,
        out_shape=jax.ShapeDtypeStruct((N, H), jnp.float32),
    )(M, A, s, Wf, bf, bn_out_g, bn_out_b)


def kernel(x, edge_index, bn_in_g, bn_in_b, W1, b1, bn1_g, bn1_b, W2, b2,
           bn2_g, bn2_b, att_w, att_b, Wf, bf, bn_out_g, bn_out_b):
    row = edge_index[0]
    col = edge_index[1]

    P, Q = _phase0(x, bn_in_g, bn_in_b, W1, b1)

    h1, st1 = _phase1(P, Q, row, col)
    s1 = jnp.sum(st1[:, :H], axis=0)
    s2 = jnp.sum(st1[:, H:], axis=0)

    mu1 = s1 / E
    var1 = s2 / E - mu1 * mu1
    rs1 = bn1_g * jax.lax.rsqrt(var1 + 1e-5)
    W2dT = (W2 * rs1[None, :]).T
    c2 = b2 + (bn1_b - mu1 * rs1) @ W2.T

    h2, st2 = _phase2(h1, W2dT, c2)
    mu2 = jnp.sum(st2[:, 0, :], axis=0) / E
    var2 = jnp.sum(st2[:, 1, :], axis=0) / E - mu2 * mu2
    rs2 = bn2_g * jax.lax.rsqrt(var2 + 1e-5)
    sh2 = bn2_b - mu2 * rs2

    wt = att_w[0] * rs2
    ct = att_b[0] + att_w[0] @ sh2
    logits, mx = _phase2b(h2, wt)
    K = jnp.max(mx[:, 0, 0]) + ct

    e = _phase3(logits + ct, K).reshape(E)

    aff = jnp.concatenate([rs2, sh2])
    Mp, Ap, sp = _phase4(h2, col, e, aff)
    M = Mp[:N]
    A = Ap[:N]
    s = sp[:, :, 0].reshape(NPAD)[:N]

    return _phase5(M, A, s, Wf, bf, bn_out_g, bn_out_b)


# speedup vs baseline: 4.0963x; 1.3713x over previous
import functools
import jax
import jax.numpy as jnp
from jax import lax
from jax.experimental import pallas as pl
from jax.experimental.pallas import tpu as pltpu
from jax.experimental.pallas import tpu_sc as plsc

N = 10000
E = 320000
C = 128
H = 64
EB = 2560
NEB = E // EB

NC = 2
NS = 16
NW = NC * NS
EPW = E // NW
CH = 400
NCH = EPW // CH
GSUB = 5
GLEN = CH // GSUB

_HI = jax.lax.Precision.HIGHEST


def _p0_body(x_ref, g_ref, b_ref, ws_ref, wb_ref, b1_ref, p_ref, q_ref):
    x = x_ref[...]
    mu = jnp.mean(x, axis=0, keepdims=True)
    var = jnp.mean((x - mu) ** 2, axis=0, keepdims=True)
    xbn = (x - mu) * jax.lax.rsqrt(var + 1e-5) * g_ref[...][None, :] + b_ref[...][None, :]
    p_ref[...] = jnp.dot(xbn, ws_ref[...], preferred_element_type=jnp.float32,
                         precision=_HI) + b1_ref[...][None, :]
    q_ref[...] = jnp.dot(xbn, wb_ref[...], preferred_element_type=jnp.float32,
                         precision=_HI)


def _phase0(x, bn_in_g, bn_in_b, W1, b1):
    Wa = W1[:, :C]
    Wb = W1[:, C:]
    WsT = (Wa + Wb).T
    WbT = Wb.T
    return pl.pallas_call(
        _p0_body,
        out_shape=[jax.ShapeDtypeStruct((N, H), jnp.float32),
                   jax.ShapeDtypeStruct((N, H), jnp.float32)],
    )(x, bn_in_g, bn_in_b, WsT, WbT, b1)


def _p1_body(p_hbm, q_hbm, row_hbm, col_hbm, h1_hbm, st_hbm,
             idxr, idxc, pbuf, qbuf, hbuf, stv, semr, semc):
    wid = lax.axis_index("s") * NC + lax.axis_index("c")
    ebase = wid * EPW

    def chunk(ci, acc):
        base = ebase + ci * CH
        pltpu.sync_copy(row_hbm.at[pl.ds(base, CH)], idxr)
        pltpu.sync_copy(col_hbm.at[pl.ds(base, CH)], idxc)
        cps = []
        for j in range(GSUB):
            sl = pl.ds(j * GLEN, GLEN)
            cps.append(pltpu.async_copy(p_hbm.at[idxr.at[sl]], pbuf.at[sl], semr))
            cps.append(pltpu.async_copy(q_hbm.at[idxc.at[sl]], qbuf.at[sl], semc))
        for cp in cps:
            cp.wait()

        def rowfn(r, acc):
            out = []
            for k in range(4):
                sl = pl.ds(k * 16, 16)
                h = jnp.maximum(pbuf[r, sl] - qbuf[r, sl], 0.0)
                hbuf[r, sl] = h
                out.append(acc[k] + h)
                out.append(acc[4 + k] + h * h)
            return (out[0], out[2], out[4], out[6], out[1], out[3], out[5], out[7])

        acc = lax.fori_loop(0, CH, rowfn, acc)
        pltpu.sync_copy(hbuf, h1_hbm.at[pl.ds(base, CH)])
        return acc

    zero = jnp.zeros((16,), jnp.float32)
    acc = lax.fori_loop(0, NCH, chunk, (zero,) * 8)
    for k in range(4):
        stv[pl.ds(k * 16, 16)] = acc[k]
        stv[pl.ds(64 + k * 16, 16)] = acc[4 + k]
    pltpu.sync_copy(stv, st_hbm.at[wid])


def _phase1(P, Q, row, col):
    return pl.kernel(
        _p1_body,
        out_type=[jax.ShapeDtypeStruct((E, H), jnp.float32),
                  jax.ShapeDtypeStruct((NW, 2 * H), jnp.float32)],
        mesh=plsc.VectorSubcoreMesh(core_axis_name="c", subcore_axis_name="s"),
        compiler_params=pltpu.CompilerParams(use_tc_tiling_on_sc=False),
        scratch_types=[
            pltpu.VMEM((CH,), jnp.int32),
            pltpu.VMEM((CH,), jnp.int32),
            pltpu.VMEM((CH, H), jnp.float32),
            pltpu.VMEM((CH, H), jnp.float32),
            pltpu.VMEM((CH, H), jnp.float32),
            pltpu.VMEM((2 * H,), jnp.float32),
            pltpu.SemaphoreType.DMA,
            pltpu.SemaphoreType.DMA,
        ],
    )(P, Q, row, col)


def _p2_body(h1_ref, w_ref, c_ref, h2_ref, st_ref):
    h2 = jnp.maximum(jnp.dot(h1_ref[...], w_ref[...],
                             preferred_element_type=jnp.float32,
                             precision=_HI) + c_ref[...][None, :], 0.0)
    h2_ref[...] = h2
    s1 = jnp.sum(h2, axis=0, keepdims=True)
    s2 = jnp.sum(h2 * h2, axis=0, keepdims=True)
    st_ref[...] = jnp.concatenate(
        [s1, s2, jnp.zeros((6, H), jnp.float32)], axis=0)[None]


def _phase2(h1, W2dT, c2):
    return pl.pallas_call(
        _p2_body,
        grid=(NEB,),
        in_specs=[pl.BlockSpec((EB, H), lambda i: (i, 0)),
                  pl.BlockSpec((H, H), lambda i: (0, 0)),
                  pl.BlockSpec((H,), lambda i: (0,))],
        out_specs=[pl.BlockSpec((EB, H), lambda i: (i, 0)),
                   pl.BlockSpec((1, 8, H), lambda i: (i, 0, 0))],
        out_shape=[jax.ShapeDtypeStruct((E, H), jnp.float32),
                   jax.ShapeDtypeStruct((NEB, 8, H), jnp.float32)],
    )(h1, W2dT, c2)


def _p2b_body(h2_ref, wt_ref, l_ref, mx_ref):
    h3 = h2_ref[...].reshape(EB // 128, 128, H)
    l = jnp.sum(h3 * wt_ref[...][None, None, :], axis=-1)
    l_ref[...] = l[None]
    mx_ref[...] = jnp.full((1, 8, H), jnp.max(l), jnp.float32)


def _phase2b(h2, wt):
    return pl.pallas_call(
        _p2b_body,
        grid=(NEB,),
        in_specs=[pl.BlockSpec((EB, H), lambda i: (i, 0)),
                  pl.BlockSpec((H,), lambda i: (0,))],
        out_specs=[pl.BlockSpec((1, EB // 128, 128), lambda i: (i, 0, 0)),
                   pl.BlockSpec((1, 8, H), lambda i: (i, 0, 0))],
        out_shape=[jax.ShapeDtypeStruct((NEB, EB // 128, 128), jnp.float32),
                   jax.ShapeDtypeStruct((NEB, 8, H), jnp.float32)],
    )(h2, wt)


def _p3_body(l_ref, k_ref, e_ref):
    e_ref[...] = jnp.exp(l_ref[...] - k_ref[0, 0])


def _phase3(logits, K):
    return pl.pallas_call(
        _p3_body,
        grid=(NEB,),
        in_specs=[pl.BlockSpec((1, EB // 128, 128), lambda i: (i, 0, 0)),
                  pl.BlockSpec((1, 1), lambda i: (0, 0))],
        out_specs=pl.BlockSpec((1, EB // 128, 128), lambda i: (i, 0, 0)),
        out_shape=jax.ShapeDtypeStruct((NEB, EB // 128, 128), jnp.float32),
    )(logits, K.reshape(1, 1))


NPB = 313
NPAD = NPB * NW
SCH = 400
NG = SCH // 16
NSCH = E // SCH
GMAX = SCH + 16


def _p4_body(h2_hbm, col_hbm, e_hbm, aff_hbm, m_hbm, a_hbm, s_hbm,
             colv0, colv1, ev0, ev1, idl0, idl1, cll0, cll1, ell0, ell1,
             gbuf0, gbuf1, macc, aacc, sacc, affv, semc, seme, semg0, semg1):
    wid = lax.axis_index("s") * NC + lax.axis_index("c")
    nlo = wid * NPB
    iota = lax.iota(jnp.int32, 16)

    pltpu.sync_copy(aff_hbm, affv)
    rs = [affv[pl.ds(k * 16, 16)] for k in range(4)]
    sh = [affv[pl.ds(H + k * 16, 16)] for k in range(4)]

    neg = jnp.full((16,), -jnp.inf, jnp.float32)
    zf = jnp.zeros((16,), jnp.float32)
    zi = jnp.zeros((16,), jnp.int32)

    def initrow(i, _):
        for k in range(4):
            sl = pl.ds(k * 16, 16)
            macc[i, sl] = neg
            aacc[i, sl] = zf
        sacc[i, :] = zf
        return 0
    lax.fori_loop(0, NPB, initrow, 0)
    for i in range(GMAX // 16):
        idl0[pl.ds(i * 16, 16)] = zi
        idl1[pl.ds(i * 16, 16)] = zi

    def issue_pref(ci, colv, ev):
        base = jnp.minimum(ci, NSCH - 1) * SCH
        pltpu.async_copy(col_hbm.at[pl.ds(base, SCH)], colv, semc)
        pltpu.async_copy(e_hbm.at[pl.ds(base, SCH)], ev, seme)

    def wait_pref(colv, ev):
        pltpu.make_async_copy(col_hbm.at[pl.ds(0, SCH)], colv, semc).wait()
        pltpu.make_async_copy(e_hbm.at[pl.ds(0, SCH)], ev, seme).wait()

    def scan(colv, ev, idl, cll, ell, base):
        def grp(g, cnt):
            sl16 = pl.ds(g * 16, 16)
            c16 = colv[sl16]
            msk = (c16 >= nlo) & (c16 < nlo + NPB)
            dst = pl.ds(cnt, 16)
            plsc.store_compressed(idl.at[dst], iota + (base + g * 16), mask=msk)
            plsc.store_compressed(cll.at[dst], c16, mask=msk)
            plsc.store_compressed(ell.at[dst], ev[sl16], mask=msk)
            return cnt + plsc.all_reduce_population_count(msk)[0]
        return lax.fori_loop(0, NG, grp, 0)

    def fire(idl, gbuf, m, semg):
        def f(j, _):
            sl = pl.ds(pl.multiple_of(j * 16, 16), 16)
            pltpu.async_copy(h2_hbm.at[idl.at[sl]], gbuf.at[sl], semg)
            return 0
        lax.fori_loop(0, lax.div(m + 15, 16), f, 0)

    def drain_rmw(cll, ell, gbuf, m, semg):
        def d(j, _):
            sl = pl.ds(pl.multiple_of(j * 16, 16), 16)
            pltpu.make_async_copy(h2_hbm.at[pl.ds(0, 16)], gbuf.at[sl], semg).wait()
            return 0
        lax.fori_loop(0, lax.div(m + 15, 16), d, 0)

        def edge(r, _):
            ln = cll[pl.ds(r, 16)][0] - nlo
            ee = ell[pl.ds(r, 16)][0]
            for k in range(4):
                sl = pl.ds(k * 16, 16)
                v = gbuf[r, sl] * rs[k] + sh[k]
                macc[ln, sl] = jnp.maximum(macc[ln, sl], v)
                aacc[ln, sl] = aacc[ln, sl] + v * ee
            sacc[ln, :] = sacc[ln, :] + ee
            return 0
        lax.fori_loop(0, m, edge, 0)

    issue_pref(0, colv0, ev0)

    def body(t, m_prev):
        ci = 2 * t
        wait_pref(colv0, ev0)
        issue_pref(ci + 1, colv1, ev1)
        m0 = scan(colv0, ev0, idl0, cll0, ell0, ci * SCH)
        fire(idl0, gbuf0, m0, semg0)
        drain_rmw(cll1, ell1, gbuf1, m_prev, semg1)

        wait_pref(colv1, ev1)
        issue_pref(ci + 2, colv0, ev0)
        m1 = scan(colv1, ev1, idl1, cll1, ell1, (ci + 1) * SCH)
        fire(idl1, gbuf1, m1, semg1)
        drain_rmw(cll0, ell0, gbuf0, m0, semg0)
        return m1

    m_last = lax.fori_loop(0, NSCH // 2, body, 0)
    wait_pref(colv0, ev0)
    drain_rmw(cll1, ell1, gbuf1, m_last, semg1)

    pltpu.sync_copy(macc, m_hbm.at[pl.ds(nlo, NPB)])
    pltpu.sync_copy(aacc, a_hbm.at[pl.ds(nlo, NPB)])
    pltpu.sync_copy(sacc, s_hbm.at[wid])


def _phase4(h2, col, e, aff):
    return pl.kernel(
        _p4_body,
        out_type=[jax.ShapeDtypeStruct((NPAD, H), jnp.float32),
                  jax.ShapeDtypeStruct((NPAD, H), jnp.float32),
                  jax.ShapeDtypeStruct((NW, NPB, 16), jnp.float32)],
        mesh=plsc.VectorSubcoreMesh(core_axis_name="c", subcore_axis_name="s"),
        compiler_params=pltpu.CompilerParams(use_tc_tiling_on_sc=False,
                                             needs_layout_passes=False),
        scratch_types=[
            pltpu.VMEM((SCH,), jnp.int32),
            pltpu.VMEM((SCH,), jnp.int32),
            pltpu.VMEM((SCH,), jnp.float32),
            pltpu.VMEM((SCH,), jnp.float32),
            pltpu.VMEM((GMAX,), jnp.int32),
            pltpu.VMEM((GMAX,), jnp.int32),
            pltpu.VMEM((GMAX,), jnp.int32),
            pltpu.VMEM((GMAX,), jnp.int32),
            pltpu.VMEM((GMAX,), jnp.float32),
            pltpu.VMEM((GMAX,), jnp.float32),
            pltpu.VMEM((GMAX, H), jnp.float32),
            pltpu.VMEM((GMAX, H), jnp.float32),
            pltpu.VMEM((NPB, H), jnp.float32),
            pltpu.VMEM((NPB, H), jnp.float32),
            pltpu.VMEM((NPB, 16), jnp.float32),
            pltpu.VMEM((2 * H,), jnp.float32),
            pltpu.SemaphoreType.DMA,
            pltpu.SemaphoreType.DMA,
            pltpu.SemaphoreType.DMA,
            pltpu.SemaphoreType.DMA,
        ],
    )(h2, col, e, aff)


def _p5_body(m_ref, a_ref, s_ref, wf_ref, bf_ref, g_ref, b_ref, o_ref):
    M = m_ref[...]
    M = jnp.where(M == -jnp.inf, 0.0, M)
    att = a_ref[...] / (s_ref[...][:, None] + 1e-16)
    out = (jnp.dot(M, wf_ref[...][:, :H].T, preferred_element_type=jnp.float32,
                   precision=_HI)
           + jnp.dot(att, wf_ref[...][:, H:].T, preferred_element_type=jnp.float32,
                     precision=_HI) + bf_ref[...][None, :])
    mu = jnp.mean(out, axis=0, keepdims=True)
    var = jnp.mean((out - mu) ** 2, axis=0, keepdims=True)
    o_ref[...] = jnp.maximum(
        (out - mu) * jax.lax.rsqrt(var + 1e-5) * g_ref[...][None, :]
        + b_ref[...][None, :], 0.0)


def _phase5(M, A, s, Wf, bf, bn_out_g, bn_out_b):
    return pl.pallas_call(
        _p5_body,
        out_shape=jax.ShapeDtypeStruct((N, H), jnp.float32),
    )(M, A, s, Wf, bf, bn_out_g, bn_out_b)


def kernel(x, edge_index, bn_in_g, bn_in_b, W1, b1, bn1_g, bn1_b, W2, b2,
           bn2_g, bn2_b, att_w, att_b, Wf, bf, bn_out_g, bn_out_b):
    row = edge_index[0]
    col = edge_index[1]

    P, Q = _phase0(x, bn_in_g, bn_in_b, W1, b1)

    h1, st1 = _phase1(P, Q, row, col)
    s1 = jnp.sum(st1[:, :H], axis=0)
    s2 = jnp.sum(st1[:, H:], axis=0)

    mu1 = s1 / E
    var1 = s2 / E - mu1 * mu1
    rs1 = bn1_g * jax.lax.rsqrt(var1 + 1e-5)
    W2dT = (W2 * rs1[None, :]).T
    c2 = b2 + (bn1_b - mu1 * rs1) @ W2.T

    h2, st2 = _phase2(h1, W2dT, c2)
    mu2 = jnp.sum(st2[:, 0, :], axis=0) / E
    var2 = jnp.sum(st2[:, 1, :], axis=0) / E - mu2 * mu2
    rs2 = bn2_g * jax.lax.rsqrt(var2 + 1e-5)
    sh2 = bn2_b - mu2 * rs2

    wt = att_w[0] * rs2
    ct = att_b[0] + att_w[0] @ sh2
    logits, mx = _phase2b(h2, wt)
    K = jnp.max(mx[:, 0, 0]) + ct

    e = _phase3(logits + ct, K).reshape(E)

    aff = jnp.concatenate([rs2, sh2])
    Mp, Ap, sp = _phase4(h2, col, e, aff)
    M = Mp[:N]
    A = Ap[:N]
    s = sp[:, :, 0].reshape(NPAD)[:N]

    return _phase5(M, A, s, Wf, bf, bn_out_g, bn_out_b)


# speedup vs baseline: 4.7073x; 1.1492x over previous
import functools
import jax
import jax.numpy as jnp
from jax import lax
from jax.experimental import pallas as pl
from jax.experimental.pallas import tpu as pltpu
from jax.experimental.pallas import tpu_sc as plsc

N = 10000
E = 320000
C = 128
H = 64
EB = 2560
NEB = E // EB

NC = 2
NS = 16
NW = NC * NS
EPW = E // NW
CH = 400
NCH = EPW // CH
GSUB = 5
GLEN = CH // GSUB

_HI = jax.lax.Precision.HIGHEST


def _p0_body(x_ref, g_ref, b_ref, ws_ref, wb_ref, b1_ref, p_ref, q_ref):
    x = x_ref[...]
    mu = jnp.mean(x, axis=0, keepdims=True)
    var = jnp.mean((x - mu) ** 2, axis=0, keepdims=True)
    xbn = (x - mu) * jax.lax.rsqrt(var + 1e-5) * g_ref[...][None, :] + b_ref[...][None, :]
    p_ref[...] = jnp.dot(xbn, ws_ref[...], preferred_element_type=jnp.float32,
                         precision=_HI) + b1_ref[...][None, :]
    q_ref[...] = jnp.dot(xbn, wb_ref[...], preferred_element_type=jnp.float32,
                         precision=_HI)


def _phase0(x, bn_in_g, bn_in_b, W1, b1):
    Wa = W1[:, :C]
    Wb = W1[:, C:]
    WsT = (Wa + Wb).T
    WbT = Wb.T
    return pl.pallas_call(
        _p0_body,
        out_shape=[jax.ShapeDtypeStruct((N, H), jnp.float32),
                   jax.ShapeDtypeStruct((N, H), jnp.float32)],
    )(x, bn_in_g, bn_in_b, WsT, WbT, b1)


def _p1_body(p_hbm, q_hbm, row_hbm, col_hbm, h1_hbm, st_hbm,
             idxr, idxc, pbuf, qbuf, hbuf, stv, semr, semc):
    wid = lax.axis_index("s") * NC + lax.axis_index("c")
    ebase = wid * EPW

    def chunk(ci, acc):
        base = ebase + ci * CH
        pltpu.sync_copy(row_hbm.at[pl.ds(base, CH)], idxr)
        pltpu.sync_copy(col_hbm.at[pl.ds(base, CH)], idxc)
        cps = []
        for j in range(GSUB):
            sl = pl.ds(j * GLEN, GLEN)
            cps.append(pltpu.async_copy(p_hbm.at[idxr.at[sl]], pbuf.at[sl], semr))
            cps.append(pltpu.async_copy(q_hbm.at[idxc.at[sl]], qbuf.at[sl], semc))
        for cp in cps:
            cp.wait()

        def rowfn(r, acc):
            out = []
            for k in range(4):
                sl = pl.ds(k * 16, 16)
                h = jnp.maximum(pbuf[r, sl] - qbuf[r, sl], 0.0)
                hbuf[r, sl] = h
                out.append(acc[k] + h)
                out.append(acc[4 + k] + h * h)
            return (out[0], out[2], out[4], out[6], out[1], out[3], out[5], out[7])

        acc = lax.fori_loop(0, CH, rowfn, acc)
        pltpu.sync_copy(hbuf, h1_hbm.at[pl.ds(base, CH)])
        return acc

    zero = jnp.zeros((16,), jnp.float32)
    acc = lax.fori_loop(0, NCH, chunk, (zero,) * 8)
    for k in range(4):
        stv[pl.ds(k * 16, 16)] = acc[k]
        stv[pl.ds(64 + k * 16, 16)] = acc[4 + k]
    pltpu.sync_copy(stv, st_hbm.at[wid])


def _phase1(P, Q, row, col):
    return pl.kernel(
        _p1_body,
        out_type=[jax.ShapeDtypeStruct((E, H), jnp.float32),
                  jax.ShapeDtypeStruct((NW, 2 * H), jnp.float32)],
        mesh=plsc.VectorSubcoreMesh(core_axis_name="c", subcore_axis_name="s"),
        compiler_params=pltpu.CompilerParams(use_tc_tiling_on_sc=False),
        scratch_types=[
            pltpu.VMEM((CH,), jnp.int32),
            pltpu.VMEM((CH,), jnp.int32),
            pltpu.VMEM((CH, H), jnp.float32),
            pltpu.VMEM((CH, H), jnp.float32),
            pltpu.VMEM((CH, H), jnp.float32),
            pltpu.VMEM((2 * H,), jnp.float32),
            pltpu.SemaphoreType.DMA,
            pltpu.SemaphoreType.DMA,
        ],
    )(P, Q, row, col)


E2 = E // 2
EB2 = EB // 2


def _p2_body(h1_ref, w_ref, c_ref, h2_ref, st_ref):
    h2 = jnp.maximum(jnp.dot(h1_ref[...], w_ref[...],
                             preferred_element_type=jnp.float32,
                             precision=_HI) + c_ref[...][None, :], 0.0)
    h2_ref[...] = h2
    s1 = jnp.sum(h2, axis=0, keepdims=True)
    s2 = jnp.sum(h2 * h2, axis=0, keepdims=True)
    st_ref[...] = jnp.concatenate(
        [s1, s2, jnp.zeros((6, 2 * H), jnp.float32)], axis=0)[None]


def _phase2(h1p, W128, c2c):
    return pl.pallas_call(
        _p2_body,
        grid=(NEB,),
        in_specs=[pl.BlockSpec((EB2, 2 * H), lambda i: (i, 0)),
                  pl.BlockSpec((2 * H, 2 * H), lambda i: (0, 0)),
                  pl.BlockSpec((2 * H,), lambda i: (0,))],
        out_specs=[pl.BlockSpec((EB2, 2 * H), lambda i: (i, 0)),
                   pl.BlockSpec((1, 8, 2 * H), lambda i: (i, 0, 0))],
        out_shape=[jax.ShapeDtypeStruct((E2, 2 * H), jnp.float32),
                   jax.ShapeDtypeStruct((NEB, 8, 2 * H), jnp.float32)],
    )(h1p, W128, c2c)


def _p2b_body(h2_ref, we_ref, wo_ref, l_ref, mx_ref):
    h3 = h2_ref[...].reshape(EB2 // 128, 128, 2 * H)
    le = jnp.sum(h3 * we_ref[...][None, None, :], axis=-1)
    lo = jnp.sum(h3 * wo_ref[...][None, None, :], axis=-1)
    l = jnp.concatenate([le, lo], axis=0)
    l_ref[...] = l[None]
    mx_ref[...] = jnp.full((1, 8, H), jnp.max(l), jnp.float32)


def _phase2b(h2p, wte, wto):
    return pl.pallas_call(
        _p2b_body,
        grid=(NEB,),
        in_specs=[pl.BlockSpec((EB2, 2 * H), lambda i: (i, 0)),
                  pl.BlockSpec((2 * H,), lambda i: (0,)),
                  pl.BlockSpec((2 * H,), lambda i: (0,))],
        out_specs=[pl.BlockSpec((1, EB // 128, 128), lambda i: (i, 0, 0)),
                   pl.BlockSpec((1, 8, H), lambda i: (i, 0, 0))],
        out_shape=[jax.ShapeDtypeStruct((NEB, EB // 128, 128), jnp.float32),
                   jax.ShapeDtypeStruct((NEB, 8, H), jnp.float32)],
    )(h2p, wte, wto)


def _p3_body(l_ref, k_ref, e_ref):
    e_ref[...] = jnp.exp(l_ref[...] - k_ref[0, 0])


def _phase3(logits, K):
    return pl.pallas_call(
        _p3_body,
        grid=(NEB,),
        in_specs=[pl.BlockSpec((1, EB // 128, 128), lambda i: (i, 0, 0)),
                  pl.BlockSpec((1, 1), lambda i: (0, 0))],
        out_specs=pl.BlockSpec((1, EB // 128, 128), lambda i: (i, 0, 0)),
        out_shape=jax.ShapeDtypeStruct((NEB, EB // 128, 128), jnp.float32),
    )(logits, K.reshape(1, 1))


NPB = 313
NPAD = NPB * NW
SCH = 400
NG = SCH // 16
NSCH = E // SCH
GMAX = SCH + 16


def _p4_body(h2_hbm, col_hbm, e_hbm, aff_hbm, m_hbm, a_hbm, s_hbm,
             colv0, colv1, ev0, ev1, idl0, idl1, cll0, cll1, ell0, ell1,
             gbuf0, gbuf1, macc, aacc, sacc, affv, semc, seme, semg0, semg1):
    wid = lax.axis_index("s") * NC + lax.axis_index("c")
    nlo = wid * NPB
    iota = lax.iota(jnp.int32, 16)

    pltpu.sync_copy(aff_hbm, affv)
    rs = [affv[pl.ds(k * 16, 16)] for k in range(4)]
    sh = [affv[pl.ds(H + k * 16, 16)] for k in range(4)]

    neg = jnp.full((16,), -jnp.inf, jnp.float32)
    zf = jnp.zeros((16,), jnp.float32)
    zi = jnp.zeros((16,), jnp.int32)

    def initrow(i, _):
        for k in range(4):
            sl = pl.ds(k * 16, 16)
            macc[i, sl] = neg
            aacc[i, sl] = zf
        sacc[i, :] = zf
        return 0
    lax.fori_loop(0, NPB, initrow, 0)
    for i in range(GMAX // 16):
        idl0[pl.ds(i * 16, 16)] = zi
        idl1[pl.ds(i * 16, 16)] = zi

    def issue_pref(ci, colv, ev):
        base = jnp.minimum(ci, NSCH - 1) * SCH
        pltpu.async_copy(col_hbm.at[pl.ds(base, SCH)], colv, semc)
        pltpu.async_copy(e_hbm.at[pl.ds(base, SCH)], ev, seme)

    def wait_pref(colv, ev):
        pltpu.make_async_copy(col_hbm.at[pl.ds(0, SCH)], colv, semc).wait()
        pltpu.make_async_copy(e_hbm.at[pl.ds(0, SCH)], ev, seme).wait()

    def scan(colv, ev, idl, cll, ell, base):
        def grp(g, cnt):
            sl16 = pl.ds(g * 16, 16)
            c16 = colv[sl16]
            msk = (c16 >= nlo) & (c16 < nlo + NPB)
            dst = pl.ds(cnt, 16)
            plsc.store_compressed(idl.at[dst], iota + (base + g * 16), mask=msk)
            plsc.store_compressed(cll.at[dst], c16, mask=msk)
            plsc.store_compressed(ell.at[dst], ev[sl16], mask=msk)
            return cnt + plsc.all_reduce_population_count(msk)[0]
        return lax.fori_loop(0, NG, grp, 0)

    def fire(idl, gbuf, m, semg):
        def f(j, _):
            sl = pl.ds(pl.multiple_of(j * 16, 16), 16)
            pltpu.async_copy(h2_hbm.at[idl.at[sl]], gbuf.at[sl], semg)
            return 0
        lax.fori_loop(0, lax.div(m + 15, 16), f, 0)

    def drain_rmw(cll, ell, gbuf, m, semg):
        def d(j, _):
            sl = pl.ds(pl.multiple_of(j * 16, 16), 16)
            pltpu.make_async_copy(h2_hbm.at[pl.ds(0, 16)], gbuf.at[sl], semg).wait()
            return 0
        lax.fori_loop(0, lax.div(m + 15, 16), d, 0)

        def edge(r, _):
            ln = cll[pl.ds(r, 16)][0] - nlo
            ee = ell[pl.ds(r, 16)][0]
            for k in range(4):
                sl = pl.ds(k * 16, 16)
                v = gbuf[r, sl] * rs[k] + sh[k]
                macc[ln, sl] = jnp.maximum(macc[ln, sl], v)
                aacc[ln, sl] = aacc[ln, sl] + v * ee
            sacc[ln, :] = sacc[ln, :] + ee
            return 0
        lax.fori_loop(0, m, edge, 0)

    issue_pref(0, colv0, ev0)

    def body(t, m_prev):
        ci = 2 * t
        wait_pref(colv0, ev0)
        issue_pref(ci + 1, colv1, ev1)
        m0 = scan(colv0, ev0, idl0, cll0, ell0, ci * SCH)
        fire(idl0, gbuf0, m0, semg0)
        drain_rmw(cll1, ell1, gbuf1, m_prev, semg1)

        wait_pref(colv1, ev1)
        issue_pref(ci + 2, colv0, ev0)
        m1 = scan(colv1, ev1, idl1, cll1, ell1, (ci + 1) * SCH)
        fire(idl1, gbuf1, m1, semg1)
        drain_rmw(cll0, ell0, gbuf0, m0, semg0)
        return m1

    m_last = lax.fori_loop(0, NSCH // 2, body, 0)
    wait_pref(colv0, ev0)
    drain_rmw(cll1, ell1, gbuf1, m_last, semg1)

    pltpu.sync_copy(macc, m_hbm.at[pl.ds(nlo, NPB)])
    pltpu.sync_copy(aacc, a_hbm.at[pl.ds(nlo, NPB)])
    pltpu.sync_copy(sacc, s_hbm.at[wid])


def _phase4(h2, col, e, aff):
    return pl.kernel(
        _p4_body,
        out_type=[jax.ShapeDtypeStruct((NPAD, H), jnp.float32),
                  jax.ShapeDtypeStruct((NPAD, H), jnp.float32),
                  jax.ShapeDtypeStruct((NW, NPB, 16), jnp.float32)],
        mesh=plsc.VectorSubcoreMesh(core_axis_name="c", subcore_axis_name="s"),
        compiler_params=pltpu.CompilerParams(use_tc_tiling_on_sc=False,
                                             needs_layout_passes=False),
        scratch_types=[
            pltpu.VMEM((SCH,), jnp.int32),
            pltpu.VMEM((SCH,), jnp.int32),
            pltpu.VMEM((SCH,), jnp.float32),
            pltpu.VMEM((SCH,), jnp.float32),
            pltpu.VMEM((GMAX,), jnp.int32),
            pltpu.VMEM((GMAX,), jnp.int32),
            pltpu.VMEM((GMAX,), jnp.int32),
            pltpu.VMEM((GMAX,), jnp.int32),
            pltpu.VMEM((GMAX,), jnp.float32),
            pltpu.VMEM((GMAX,), jnp.float32),
            pltpu.VMEM((GMAX, H), jnp.float32),
            pltpu.VMEM((GMAX, H), jnp.float32),
            pltpu.VMEM((NPB, H), jnp.float32),
            pltpu.VMEM((NPB, H), jnp.float32),
            pltpu.VMEM((NPB, 16), jnp.float32),
            pltpu.VMEM((2 * H,), jnp.float32),
            pltpu.SemaphoreType.DMA,
            pltpu.SemaphoreType.DMA,
            pltpu.SemaphoreType.DMA,
            pltpu.SemaphoreType.DMA,
        ],
    )(h2, col, e, aff)


def _p5_body(m_ref, a_ref, s_ref, wf_ref, bf_ref, g_ref, b_ref, o_ref):
    M = m_ref[...]
    M = jnp.where(M == -jnp.inf, 0.0, M)
    att = a_ref[...] / (s_ref[...][:, None] + 1e-16)
    out = (jnp.dot(M, wf_ref[...][:, :H].T, preferred_element_type=jnp.float32,
                   precision=_HI)
           + jnp.dot(att, wf_ref[...][:, H:].T, preferred_element_type=jnp.float32,
                     precision=_HI) + bf_ref[...][None, :])
    mu = jnp.mean(out, axis=0, keepdims=True)
    var = jnp.mean((out - mu) ** 2, axis=0, keepdims=True)
    o_ref[...] = jnp.maximum(
        (out - mu) * jax.lax.rsqrt(var + 1e-5) * g_ref[...][None, :]
        + b_ref[...][None, :], 0.0)


def _phase5(M, A, s, Wf, bf, bn_out_g, bn_out_b):
    return pl.pallas_call(
        _p5_body,
        out_shape=jax.ShapeDtypeStruct((N, H), jnp.float32),
    )(M, A, s, Wf, bf, bn_out_g, bn_out_b)


def kernel(x, edge_index, bn_in_g, bn_in_b, W1, b1, bn1_g, bn1_b, W2, b2,
           bn2_g, bn2_b, att_w, att_b, Wf, bf, bn_out_g, bn_out_b):
    row = edge_index[0]
    col = edge_index[1]

    P, Q = _phase0(x, bn_in_g, bn_in_b, W1, b1)

    h1, st1 = _phase1(P, Q, row, col)
    s1 = jnp.sum(st1[:, :H], axis=0)
    s2 = jnp.sum(st1[:, H:], axis=0)

    mu1 = s1 / E
    var1 = s2 / E - mu1 * mu1
    rs1 = bn1_g * jax.lax.rsqrt(var1 + 1e-5)
    W2dT = (W2 * rs1[None, :]).T
    c2 = b2 + (bn1_b - mu1 * rs1) @ W2.T

    z64 = jnp.zeros((H, H), jnp.float32)
    W128 = jnp.concatenate(
        [jnp.concatenate([W2dT, z64], axis=1),
         jnp.concatenate([z64, W2dT], axis=1)], axis=0)
    c2c = jnp.concatenate([c2, c2])
    h1p = h1.reshape(E2, 2 * H)

    h2p, st2 = _phase2(h1p, W128, c2c)
    sum128 = jnp.sum(st2[:, 0, :], axis=0)
    sq128 = jnp.sum(st2[:, 1, :], axis=0)
    mu2 = (sum128[:H] + sum128[H:]) / E
    var2 = (sq128[:H] + sq128[H:]) / E - mu2 * mu2
    rs2 = bn2_g * jax.lax.rsqrt(var2 + 1e-5)
    sh2 = bn2_b - mu2 * rs2

    wt = att_w[0] * rs2
    ct = att_b[0] + att_w[0] @ sh2
    zH = jnp.zeros((H,), jnp.float32)
    wte = jnp.concatenate([wt, zH])
    wto = jnp.concatenate([zH, wt])
    logits, mx = _phase2b(h2p, wte, wto)
    K = jnp.max(mx[:, 0, 0]) + ct

    e3 = _phase3(logits + ct, K)
    ee = e3[:, :EB2 // 128, :].reshape(E2)
    eo = e3[:, EB2 // 128:, :].reshape(E2)
    e = jnp.stack([ee, eo], axis=-1).reshape(E)

    aff = jnp.concatenate([rs2, sh2])
    Mp, Ap, sp = _phase4(h2p.reshape(E, H), col, e, aff)
    M = Mp[:N]
    A = Ap[:N]
    s = sp[:, :, 0].reshape(NPAD)[:N]

    return _phase5(M, A, s, Wf, bf, bn_out_g, bn_out_b)
